# baseline (speedup 1.0000x reference)
import functools

import jax
import jax.numpy as jnp
from jax.experimental import pallas as pl
from jax.experimental.pallas import tpu as pltpu


FEA = 128
EDGE = 16
NHEAD = 3


def _leaky(x):
    return jnp.where(x >= 0, x, 0.01 * x)


def _edge_block_kernel(nparams, self_f_ref, nbr_f_ref, ef_ref, *rest):
    param_refs = rest[:nparams]
    logit_ref, msg_ref = rest[nparams:]
    p = [r[...] for r in param_refs]
    it = iter(p)

    def nxt():
        return next(it)

    self_f = self_f_ref[...]
    nbr_f = nbr_f_ref[...]
    x = ef_ref[...]
    for _ in range(4):
        W, b = nxt(), nxt()
        x = _leaky(jnp.dot(x, W, preferred_element_type=jnp.float32) + b)
    W, b = nxt(), nxt()
    ef = jnp.dot(x, W, preferred_element_type=jnp.float32) + b

    for h in range(NHEAD):
        Wg_s, Wg_n, Wg_e, bg1, Wg2, bg2 = nxt(), nxt(), nxt(), nxt(), nxt(), nxt()
        Wm_s, Wm_n, Wm_e, bm1, Wm2, bm2 = nxt(), nxt(), nxt(), nxt(), nxt(), nxt()
        gh = _leaky(
            jnp.dot(self_f, Wg_s, preferred_element_type=jnp.float32)
            + jnp.dot(nbr_f, Wg_n, preferred_element_type=jnp.float32)
            + jnp.dot(ef, Wg_e, preferred_element_type=jnp.float32)
            + bg1
        )
        logit = jnp.dot(gh, Wg2, preferred_element_type=jnp.float32) + bg2
        logit_ref[h, :] = logit[:, 0]
        mh = _leaky(
            jnp.dot(self_f, Wm_s, preferred_element_type=jnp.float32)
            + jnp.dot(nbr_f, Wm_n, preferred_element_type=jnp.float32)
            + jnp.dot(ef, Wm_e, preferred_element_type=jnp.float32)
            + bm1
        )
        msg_ref[h, :, :] = jnp.dot(mh, Wm2, preferred_element_type=jnp.float32) + bm2


def _flatten_layer_params(g):
    out = []
    for W, b in g["edge_ebd"]["fcs"]:
        out += [W, b.reshape(1, -1)]
    W, b = g["edge_ebd"]["out"]
    out += [W, b.reshape(1, -1)]
    for h in g["heads"]:
        for mlp in (h["gate"], h["msg"]):
            (W1, b1), = mlp["fcs"]
            W2, b2 = mlp["out"]
            out += [W1[:FEA], W1[FEA:2 * FEA], W1[2 * FEA:], b1.reshape(1, -1),
                    W2, b2.reshape(1, -1)]
    return out


def _edge_compute(self_f, nbr_f, edge_fea, flat_params, block_e):
    M = edge_fea.shape[0]
    grid = (M // block_e,)
    nparams = len(flat_params)

    def full(a):
        return pl.BlockSpec(a.shape, lambda i: (0,) * a.ndim)

    in_specs = [
        pl.BlockSpec((block_e, FEA), lambda i: (i, 0)),
        pl.BlockSpec((block_e, FEA), lambda i: (i, 0)),
        pl.BlockSpec((block_e, EDGE), lambda i: (i, 0)),
    ] + [full(a) for a in flat_params]
    out_specs = [
        pl.BlockSpec((NHEAD, block_e), lambda i: (0, i)),
        pl.BlockSpec((NHEAD, block_e, FEA), lambda i: (0, i, 0)),
    ]
    out_shapes = [
        jax.ShapeDtypeStruct((NHEAD, M), jnp.float32),
        jax.ShapeDtypeStruct((NHEAD, M, FEA), jnp.float32),
    ]
    logits, msgs = pl.pallas_call(
        functools.partial(_edge_block_kernel, nparams),
        grid=grid,
        in_specs=in_specs,
        out_specs=out_specs,
        out_shape=out_shapes,
    )(self_f, nbr_f, edge_fea, *flat_params)
    return logits, msgs


def _attn_pool_from(logit, msg, index, weights, pow_, num_segments):
    gate = logit[:, None]
    gmax = jax.ops.segment_max(gate, index, num_segments=num_segments)
    gate = gate - gmax[index]
    gate = (weights ** pow_) * jnp.exp(gate)
    gsum = jax.ops.segment_sum(gate, index, num_segments=num_segments)
    gate = gate / (gsum[index] + 1e-10)
    out = jax.ops.segment_sum(gate * msg, index, num_segments=num_segments)
    return out, gate


def kernel(elem_weights, elem_fea, edge_fea, self_fea_idx, nbr_fea_idx, cry_elem_idx, params):
    N = elem_fea.shape[0]
    C = 2000
    BLOCK_E = 512

    W, b = params["embedding"]
    fea = elem_fea @ W + b
    fea = jnp.concatenate([fea, elem_weights], axis=1)

    for g in params["graphs"]:
        nbr_w = elem_weights[nbr_fea_idx, :]
        nbr_f = fea[nbr_fea_idx, :]
        self_f = fea[self_fea_idx, :]
        flat = _flatten_layer_params(g)
        logits, msgs = _edge_compute(self_f, nbr_f, edge_fea, flat, BLOCK_E)
        heads = []
        for h in range(NHEAD):
            pw = g["heads"][h]["pow"]
            out, _ = _attn_pool_from(logits[h], msgs[h], self_fea_idx, nbr_w, pw, N)
            heads.append(out)
        fea = jnp.mean(jnp.stack(heads), axis=0) + fea
        mean = fea.mean(axis=0)
        var = fea.var(axis=0)
        gamma, beta = g["bn"]
        fea = (fea - mean) / jnp.sqrt(var + 1e-5) * gamma + beta

    heads = []
    gates = []
    for h in params["cry"]:
        gate_l = _leaky(fea @ h["gate"]["fcs"][0][0] + h["gate"]["fcs"][0][1])
        gate_l = gate_l @ h["gate"]["out"][0] + h["gate"]["out"][1]
        msg = _leaky(fea @ h["msg"]["fcs"][0][0] + h["msg"]["fcs"][0][1])
        msg = msg @ h["msg"]["out"][0] + h["msg"]["out"][1]
        out, gate = _attn_pool_from(gate_l[:, 0], msg, cry_elem_idx, elem_weights, h["pow"], C)
        heads.append(out)
        gates.append(gate)
    return jnp.mean(jnp.stack(heads), axis=0), jnp.stack(gates)


# speedup vs baseline: 5.4489x; 5.4489x over previous
import functools

import jax
import jax.numpy as jnp
from jax import lax
from jax.experimental import pallas as pl
from jax.experimental.pallas import tpu as pltpu
from jax.experimental.pallas import tpu_sc as plsc


FEA = 128
EDGE = 16
NHEAD = 3
NW = 32
CHUNK = 80
BLOCK_E = 512
NBLK = 1000
CRY = 2000


def _leaky(x):
    return jnp.where(x >= 0, x, 0.01 * x)


def _dot(a, b):
    return jnp.dot(a, b, preferred_element_type=jnp.float32)


def _prep_kernel(ef_ref, ew_ref, w_ref, b_ref, out_ref):
    mm = _dot(ef_ref[...], w_ref[...]) + b_ref[...]
    out_ref[...] = jnp.concatenate([mm, ew_ref[...]], axis=1)


def _prep(elem_fea, elem_weights, W, b):
    N = elem_fea.shape[0]
    return pl.pallas_call(
        _prep_kernel,
        out_shape=jax.ShapeDtypeStruct((N, FEA), jnp.float32),
    )(elem_fea, elem_weights, W, b.reshape(1, -1))


def _sc_mesh():
    return plsc.VectorSubcoreMesh(core_axis_name="c", subcore_axis_name="s",
                                  num_cores=2, num_subcores=16)


def _gather_rows_body(K, fea_hbm, sidx_hbm, nidx_hbm, sout_hbm, nout_hbm,
                      idxbuf, rowbuf, sem):
    c = lax.axis_index("c")
    s = lax.axis_index("s")
    wid = s * 2 + c
    npw = K * CHUNK
    base = wid * npw

    def run(idx_hbm, out_hbm):
        pltpu.sync_copy(idx_hbm.at[wid], idxbuf)

        def body(j, _):
            pltpu.async_copy(fea_hbm.at[idxbuf.at[j]], rowbuf, sem).wait()
            pltpu.sync_copy(rowbuf, out_hbm.at[pl.ds(base + j * CHUNK, CHUNK), :])
            return 0

        lax.fori_loop(0, K, body, 0, unroll=False)

    run(sidx_hbm, sout_hbm)
    run(nidx_hbm, nout_hbm)


def _gather_rows(fea, sidx3, nidx3):
    K = sidx3.shape[1]
    M = NW * K * CHUNK
    f = pl.kernel(
        functools.partial(_gather_rows_body, K),
        out_type=[
            jax.ShapeDtypeStruct((M, FEA), jnp.float32),
            jax.ShapeDtypeStruct((M, FEA), jnp.float32),
        ],
        mesh=_sc_mesh(),
        scratch_types=[
            pltpu.VMEM((K, CHUNK), jnp.int32),
            pltpu.VMEM((CHUNK, FEA), jnp.float32),
            pltpu.SemaphoreType.DMA,
        ],
    )
    return f(fea, sidx3, nidx3)


def _edge_block_kernel(nparams, first, self_f_ref, nbr_f_ref, ef_ref, *rest):
    if first:
        param_refs = rest[:nparams]
        w_ref, wm_ref, nw_ref = rest[nparams:]
    else:
        nw_in = rest[0]
        param_refs = rest[1:1 + nparams]
        w_ref, wm_ref = rest[1 + nparams:]
    it = iter([r[...] for r in param_refs])

    def nxt():
        return next(it)

    self_f = self_f_ref[...]
    nbr_f = nbr_f_ref[...]
    if first:
        nbr_w = nbr_f[:, FEA - 1:FEA]
        nw_ref[...] = nbr_w
    else:
        nbr_w = nw_in[...]
    lw = jnp.log(nbr_w)
    x = ef_ref[...]
    for _ in range(4):
        W, b = nxt(), nxt()
        x = _leaky(_dot(x, W) + b)
    W, b = nxt(), nxt()
    ef = _dot(x, W) + b

    for h in range(NHEAD):
        Wg_s, Wg_n, Wg_e, bg1, Wg2, bg2 = (nxt() for _ in range(6))
        Wm_s, Wm_n, Wm_e, bm1, Wm2, bm2 = (nxt() for _ in range(6))
        pw = nxt()
        gh = _leaky(_dot(self_f, Wg_s) + _dot(nbr_f, Wg_n) + _dot(ef, Wg_e) + bg1)
        logit = _dot(gh, Wg2) + bg2
        w = jnp.exp(logit + pw[0, 0] * lw)
        mh = _leaky(_dot(self_f, Wm_s) + _dot(nbr_f, Wm_n) + _dot(ef, Wm_e) + bm1)
        msg = _dot(mh, Wm2) + bm2
        w_ref[h, :] = w[:, 0]
        wm_ref[h, :, :] = w * msg


def _flatten_layer_params(g):
    out = []
    for W, b in g["edge_ebd"]["fcs"]:
        out += [W, b.reshape(1, -1)]
    W, b = g["edge_ebd"]["out"]
    out += [W, b.reshape(1, -1)]
    for h in g["heads"]:
        for mlp in (h["gate"], h["msg"]):
            (W1, b1), = mlp["fcs"]
            W2, b2 = mlp["out"]
            out += [W1[:FEA], W1[FEA:2 * FEA], W1[2 * FEA:], b1.reshape(1, -1),
                    W2, b2.reshape(1, -1)]
        out += [h["pow"].reshape(1, 1)]
    return out


def _edge_compute(self_f, nbr_f, edge_fea, nbr_w, flat_params):
    M = edge_fea.shape[0]
    grid = (M // BLOCK_E,)
    nparams = len(flat_params)
    first = nbr_w is None

    def full(a):
        return pl.BlockSpec(a.shape, lambda i: (0,) * a.ndim)

    in_specs = [
        pl.BlockSpec((BLOCK_E, FEA), lambda i: (i, 0)),
        pl.BlockSpec((BLOCK_E, FEA), lambda i: (i, 0)),
        pl.BlockSpec((BLOCK_E, EDGE), lambda i: (i, 0)),
    ]
    operands = [self_f, nbr_f, edge_fea]
    if not first:
        in_specs.append(pl.BlockSpec((BLOCK_E, 1), lambda i: (i, 0)))
        operands.append(nbr_w)
    in_specs += [full(a) for a in flat_params]
    operands += flat_params
    out_specs = [
        pl.BlockSpec((NHEAD, BLOCK_E), lambda i: (0, i)),
        pl.BlockSpec((NHEAD, BLOCK_E, FEA), lambda i: (0, i, 0)),
    ]
    out_shapes = [
        jax.ShapeDtypeStruct((NHEAD, M), jnp.float32),
        jax.ShapeDtypeStruct((NHEAD, M, FEA), jnp.float32),
    ]
    if first:
        out_specs.append(pl.BlockSpec((BLOCK_E, 1), lambda i: (i, 0)))
        out_shapes.append(jax.ShapeDtypeStruct((M, 1), jnp.float32))
    return pl.pallas_call(
        functools.partial(_edge_block_kernel, nparams, first),
        grid=grid,
        in_specs=in_specs,
        out_specs=out_specs,
        out_shape=out_shapes,
    )(*operands)


def _scatter_body(K, N, wm_hbm, w_hbm, idx_hbm, zf_hbm, zw_hbm,
                  accp_hbm, waccp_hbm, idxbuf, buf, wbuf, wout, acc, wacc):
    c = lax.axis_index("c")
    s = lax.axis_index("s")
    wid = s * 2 + c
    npw = K * CHUNK
    M = NW * npw
    base = wid * npw
    pltpu.sync_copy(idx_hbm.at[wid], idxbuf)

    for h in range(NHEAD):
        @pl.when(s == 0)
        def _zero():
            pltpu.sync_copy(zf_hbm, acc)
            pltpu.sync_copy(zw_hbm, wacc)

        plsc.subcore_barrier()

        def body(j, _):
            pltpu.sync_copy(wm_hbm.at[h, pl.ds(base + j * CHUNK, CHUNK), :], buf)
            pltpu.sync_copy(w_hbm.at[pl.ds(h * M + base + j * CHUNK, CHUNK)], wbuf)
            pltpu.sync_copy(buf, acc.at[idxbuf.at[j]], add=True)
            pltpu.sync_copy(wbuf, wacc.at[idxbuf.at[j]], add=True)
            return 0

        lax.fori_loop(0, K, body, 0, unroll=False)
        plsc.subcore_barrier()

        @pl.when(s == 0)
        def _flush():
            pltpu.sync_copy(acc, accp_hbm.at[c, h])
            pltpu.sync_copy(wacc, wout)
            pltpu.sync_copy(wout, waccp_hbm.at[pl.ds((c * NHEAD + h) * N, N)])

        if h < NHEAD - 1:
            plsc.subcore_barrier()


def _scatter(wm, w, idx3, N):
    K = idx3.shape[1]
    zf = jnp.zeros((N, FEA), jnp.float32)
    zw = jnp.zeros((N,), jnp.float32)
    f = pl.kernel(
        functools.partial(_scatter_body, K, N),
        out_type=[
            jax.ShapeDtypeStruct((2, NHEAD, N, FEA), jnp.float32),
            jax.ShapeDtypeStruct((2 * NHEAD * N,), jnp.float32),
        ],
        mesh=_sc_mesh(),
        scratch_types=[
            pltpu.VMEM((K, CHUNK), jnp.int32),
            pltpu.VMEM((CHUNK, FEA), jnp.float32),
            pltpu.VMEM((CHUNK,), jnp.float32),
            pltpu.VMEM((N,), jnp.float32),
            pltpu.VMEM_SHARED((N, FEA), jnp.float32),
            pltpu.VMEM_SHARED((N,), jnp.float32),
        ],
    )
    return f(wm, w.reshape(-1), idx3, zf, zw)


def _combine_kernel(accp_ref, waccp_ref, fea_ref, gamma_ref, beta_ref, out_ref):
    h = pl.program_id(0)
    num = accp_ref[0, 0] + accp_ref[1, 0]
    den = waccp_ref[0, 0] + waccp_ref[1, 0]
    contrib = num / (den + 1e-10)

    @pl.when(h == 0)
    def _init():
        out_ref[...] = contrib

    @pl.when(h > 0)
    def _acc():
        out_ref[...] += contrib

    @pl.when(h == NHEAD - 1)
    def _bn():
        fea = out_ref[...] * (1.0 / NHEAD) + fea_ref[...]
        m = jnp.mean(fea, axis=0, keepdims=True)
        v = jnp.mean((fea - m) ** 2, axis=0, keepdims=True)
        out_ref[...] = (fea - m) / jnp.sqrt(v + 1e-5) * gamma_ref[...] + beta_ref[...]


def _combine(accp, waccp, fea, gamma, beta):
    N = fea.shape[0]
    return pl.pallas_call(
        _combine_kernel,
        grid=(NHEAD,),
        in_specs=[
            pl.BlockSpec((2, 1, N, FEA), lambda h: (0, h, 0, 0)),
            pl.BlockSpec((2, 1, N, 1), lambda h: (0, h, 0, 0)),
            pl.BlockSpec((N, FEA), lambda h: (0, 0)),
            pl.BlockSpec((1, FEA), lambda h: (0, 0)),
            pl.BlockSpec((1, FEA), lambda h: (0, 0)),
        ],
        out_specs=pl.BlockSpec((N, FEA), lambda h: (0, 0)),
        out_shape=jax.ShapeDtypeStruct((N, FEA), jnp.float32),
    )(accp, waccp.reshape(2, NHEAD, N, 1), fea,
      gamma.reshape(1, -1), beta.reshape(1, -1))


def _cry_acc_kernel(nparams, fea_ref, ew_ref, cidx_ref, *rest):
    param_refs = rest[:nparams]
    acc_ref, wt_ref = rest[nparams:]
    i = pl.program_id(0)
    it = iter([r[...] for r in param_refs])

    def nxt():
        return next(it)

    fea = fea_ref[...]
    ew = ew_ref[...]
    lew = jnp.log(ew)
    cidx = cidx_ref[...]
    onehot = jnp.where(
        cidx == jax.lax.broadcasted_iota(jnp.int32, (NBLK, CRY), 1), 1.0, 0.0
    ).astype(jnp.float32)

    @pl.when(i == 0)
    def _init():
        acc_ref[...] = jnp.zeros_like(acc_ref)

    for h in range(NHEAD):
        Wg1, bg1, Wg2, bg2, Wm1, bm1, Wm2, bm2, pw = (nxt() for _ in range(9))
        logit = _dot(_leaky(_dot(fea, Wg1) + bg1), Wg2) + bg2
        w = jnp.exp(logit + pw[0, 0] * lew)
        msg = _dot(_leaky(_dot(fea, Wm1) + bm1), Wm2) + bm2
        vals = jnp.concatenate([w * msg, w], axis=1)
        acc_ref[h, :, :] += lax.dot_general(
            onehot, vals, (((0,), (0,)), ((), ())),
            preferred_element_type=jnp.float32)
        wt_ref[h, :, :] = w


def _cry_out_kernel(acc_ref, wt_ref, cidx_ref, crys_ref, gates_ref):
    i = pl.program_id(0)
    cidx = cidx_ref[...]
    onehot = jnp.where(
        cidx == jax.lax.broadcasted_iota(jnp.int32, (NBLK, CRY), 1), 1.0, 0.0
    ).astype(jnp.float32)

    @pl.when(i == 0)
    def _crys():
        tot = jnp.zeros((CRY, FEA), jnp.float32)
        for h in range(NHEAD):
            tot = tot + acc_ref[h, :, :FEA] / (acc_ref[h, :, FEA:] + 1e-10)
        crys_ref[...] = tot * (1.0 / NHEAD)

    for h in range(NHEAD):
        gsum = _dot(onehot, acc_ref[h, :, FEA:])
        gates_ref[h, :, :] = wt_ref[h, :, :] / (gsum + 1e-10)


def _flatten_cry_params(params):
    out = []
    for h in params["cry"]:
        (W1, b1), = h["gate"]["fcs"]
        W2, b2 = h["gate"]["out"]
        (V1, c1), = h["msg"]["fcs"]
        V2, c2 = h["msg"]["out"]
        out += [W1, b1.reshape(1, -1), W2, b2.reshape(1, -1),
                V1, c1.reshape(1, -1), V2, c2.reshape(1, -1),
                h["pow"].reshape(1, 1)]
    return out


def _crystal(fea, elem_weights, cry_idx_col, params):
    N = fea.shape[0]
    flat = _flatten_cry_params(params)
    nparams = len(flat)
    grid = (N // NBLK,)

    def full(a):
        return pl.BlockSpec(a.shape, lambda i: (0,) * a.ndim)

    acc, wt = pl.pallas_call(
        functools.partial(_cry_acc_kernel, nparams),
        grid=grid,
        in_specs=[
            pl.BlockSpec((NBLK, FEA), lambda i: (i, 0)),
            pl.BlockSpec((NBLK, 1), lambda i: (i, 0)),
            pl.BlockSpec((NBLK, 1), lambda i: (i, 0)),
        ] + [full(a) for a in flat],
        out_specs=[
            pl.BlockSpec((NHEAD, CRY, FEA + 1), lambda i: (0, 0, 0)),
            pl.BlockSpec((NHEAD, NBLK, 1), lambda i: (0, i, 0)),
        ],
        out_shape=[
            jax.ShapeDtypeStruct((NHEAD, CRY, FEA + 1), jnp.float32),
            jax.ShapeDtypeStruct((NHEAD, N, 1), jnp.float32),
        ],
    )(fea, elem_weights, cry_idx_col, *flat)

    crys, gates = pl.pallas_call(
        _cry_out_kernel,
        grid=grid,
        in_specs=[
            full(acc),
            pl.BlockSpec((NHEAD, NBLK, 1), lambda i: (0, i, 0)),
            pl.BlockSpec((NBLK, 1), lambda i: (i, 0)),
        ],
        out_specs=[
            pl.BlockSpec((CRY, FEA), lambda i: (0, 0)),
            pl.BlockSpec((NHEAD, NBLK, 1), lambda i: (0, i, 0)),
        ],
        out_shape=[
            jax.ShapeDtypeStruct((CRY, FEA), jnp.float32),
            jax.ShapeDtypeStruct((NHEAD, N, 1), jnp.float32),
        ],
    )(acc, wt, cry_idx_col)
    return crys, gates


def kernel(elem_weights, elem_fea, edge_fea, self_fea_idx, nbr_fea_idx, cry_elem_idx, params):
    N = elem_fea.shape[0]
    M = edge_fea.shape[0]
    K = M // (NW * CHUNK)

    sidx3 = self_fea_idx.astype(jnp.int32).reshape(NW, K, CHUNK)
    nidx3 = nbr_fea_idx.astype(jnp.int32).reshape(NW, K, CHUNK)

    W, b = params["embedding"]
    fea = _prep(elem_fea, elem_weights, W, b)
    nbr_w = None

    for g in params["graphs"]:
        self_f, nbr_f = _gather_rows(fea, sidx3, nidx3)
        flat = _flatten_layer_params(g)
        res = _edge_compute(self_f, nbr_f, edge_fea, nbr_w, flat)
        if nbr_w is None:
            w, wm, nbr_w = res
        else:
            w, wm = res
        accp, waccp = _scatter(wm, w, sidx3, N)
        gamma, beta = g["bn"]
        fea = _combine(accp, waccp, fea, gamma, beta)

    crys, gates = _crystal(fea, elem_weights,
                           cry_elem_idx.astype(jnp.int32).reshape(N, 1), params)
    return crys, gates


# speedup vs baseline: 5.6169x; 1.0308x over previous
import functools

import jax
import jax.numpy as jnp
from jax import lax
from jax.experimental import pallas as pl
from jax.experimental.pallas import tpu as pltpu
from jax.experimental.pallas import tpu_sc as plsc


FEA = 128
EDGE = 16
NHEAD = 3
NW = 32
CHUNK = 80
BLOCK_E = 512
NBLK = 1000
CRY = 2000


def _leaky(x):
    return jnp.where(x >= 0, x, 0.01 * x)


def _dot(a, b):
    return jnp.dot(a, b, preferred_element_type=jnp.float32)


def _prep_kernel(ef_ref, ew_ref, w_ref, b_ref, out_ref):
    mm = _dot(ef_ref[...], w_ref[...]) + b_ref[...]
    out_ref[...] = jnp.concatenate([mm, ew_ref[...]], axis=1)


def _prep(elem_fea, elem_weights, W, b):
    N = elem_fea.shape[0]
    return pl.pallas_call(
        _prep_kernel,
        out_shape=jax.ShapeDtypeStruct((N, FEA), jnp.float32),
    )(elem_fea, elem_weights, W, b.reshape(1, -1))


def _sc_mesh():
    return plsc.VectorSubcoreMesh(core_axis_name="c", subcore_axis_name="s",
                                  num_cores=2, num_subcores=16)


def _gather_rows_body(K, fea_hbm, sidx_hbm, nidx_hbm, sout_hbm, nout_hbm,
                      idxbuf, rowbuf, sem):
    c = lax.axis_index("c")
    s = lax.axis_index("s")
    wid = s * 2 + c
    npw = K * CHUNK
    base = wid * npw

    def run(idx_hbm, out_hbm):
        pltpu.sync_copy(idx_hbm.at[wid], idxbuf)

        def body(j, _):
            pltpu.async_copy(fea_hbm.at[idxbuf.at[j]], rowbuf, sem).wait()
            pltpu.sync_copy(rowbuf, out_hbm.at[pl.ds(base + j * CHUNK, CHUNK), :])
            return 0

        lax.fori_loop(0, K, body, 0, unroll=False)

    run(sidx_hbm, sout_hbm)
    run(nidx_hbm, nout_hbm)


def _gather_rows(fea, sidx3, nidx3):
    K = sidx3.shape[1]
    M = NW * K * CHUNK
    f = pl.kernel(
        functools.partial(_gather_rows_body, K),
        out_type=[
            jax.ShapeDtypeStruct((M, FEA), jnp.float32),
            jax.ShapeDtypeStruct((M, FEA), jnp.float32),
        ],
        mesh=_sc_mesh(),
        scratch_types=[
            pltpu.VMEM((K, CHUNK), jnp.int32),
            pltpu.VMEM((CHUNK, FEA), jnp.float32),
            pltpu.SemaphoreType.DMA,
        ],
    )
    return f(fea, sidx3, nidx3)


def _edge_block_kernel(nparams, first, self_f_ref, nbr_f_ref, ef_ref, *rest):
    if first:
        param_refs = rest[:nparams]
        w_ref, wm_ref, nw_ref = rest[nparams:]
    else:
        nw_in = rest[0]
        param_refs = rest[1:1 + nparams]
        w_ref, wm_ref = rest[1 + nparams:]
    it = iter([r[...] for r in param_refs])

    def nxt():
        return next(it)

    self_f = self_f_ref[...]
    nbr_f = nbr_f_ref[...]
    if first:
        nbr_w = nbr_f[:, FEA - 1:FEA]
        nw_ref[...] = nbr_w
    else:
        nbr_w = nw_in[...]
    lw = jnp.log(nbr_w)
    x = ef_ref[...]
    for _ in range(4):
        W, b = nxt(), nxt()
        x = _leaky(_dot(x, W) + b)
    W, b = nxt(), nxt()
    ef = _dot(x, W) + b

    zs = []
    msgs = []
    for h in range(NHEAD):
        Wg_s, Wg_n, Wg_e, bg1, Wg2, bg2 = (nxt() for _ in range(6))
        Wm_s, Wm_n, Wm_e, bm1, Wm2, bm2 = (nxt() for _ in range(6))
        pw = nxt()
        gh = _leaky(_dot(self_f, Wg_s) + _dot(nbr_f, Wg_n) + _dot(ef, Wg_e) + bg1)
        logit = _dot(gh, Wg2) + bg2
        zs.append(logit + pw[0, 0] * lw)
        mh = _leaky(_dot(self_f, Wm_s) + _dot(nbr_f, Wm_n) + _dot(ef, Wm_e) + bm1)
        msgs.append(_dot(mh, Wm2) + bm2)
    wall = jnp.exp(jnp.concatenate(zs, axis=1))
    for h in range(NHEAD):
        w = wall[:, h:h + 1]
        w_ref[h, :] = w[:, 0]
        wm_ref[h, :, :] = w * msgs[h]


def _flatten_layer_params(g):
    out = []
    for W, b in g["edge_ebd"]["fcs"]:
        out += [W, b.reshape(1, -1)]
    W, b = g["edge_ebd"]["out"]
    out += [W, b.reshape(1, -1)]
    for h in g["heads"]:
        for mlp in (h["gate"], h["msg"]):
            (W1, b1), = mlp["fcs"]
            W2, b2 = mlp["out"]
            out += [W1[:FEA], W1[FEA:2 * FEA], W1[2 * FEA:], b1.reshape(1, -1),
                    W2, b2.reshape(1, -1)]
        out += [h["pow"].reshape(1, 1)]
    return out


def _edge_compute(self_f, nbr_f, edge_fea, nbr_w, flat_params):
    M = edge_fea.shape[0]
    grid = (M // BLOCK_E,)
    nparams = len(flat_params)
    first = nbr_w is None

    def full(a):
        return pl.BlockSpec(a.shape, lambda i: (0,) * a.ndim)

    in_specs = [
        pl.BlockSpec((BLOCK_E, FEA), lambda i: (i, 0)),
        pl.BlockSpec((BLOCK_E, FEA), lambda i: (i, 0)),
        pl.BlockSpec((BLOCK_E, EDGE), lambda i: (i, 0)),
    ]
    operands = [self_f, nbr_f, edge_fea]
    if not first:
        in_specs.append(pl.BlockSpec((BLOCK_E, 1), lambda i: (i, 0)))
        operands.append(nbr_w)
    in_specs += [full(a) for a in flat_params]
    operands += flat_params
    out_specs = [
        pl.BlockSpec((NHEAD, BLOCK_E), lambda i: (0, i)),
        pl.BlockSpec((NHEAD, BLOCK_E, FEA), lambda i: (0, i, 0)),
    ]
    out_shapes = [
        jax.ShapeDtypeStruct((NHEAD, M), jnp.float32),
        jax.ShapeDtypeStruct((NHEAD, M, FEA), jnp.float32),
    ]
    if first:
        out_specs.append(pl.BlockSpec((BLOCK_E, 1), lambda i: (i, 0)))
        out_shapes.append(jax.ShapeDtypeStruct((M, 1), jnp.float32))
    return pl.pallas_call(
        functools.partial(_edge_block_kernel, nparams, first),
        grid=grid,
        in_specs=in_specs,
        out_specs=out_specs,
        out_shape=out_shapes,
    )(*operands)


def _scatter_body(K, N, wm_hbm, w_hbm, idx_hbm, zf_hbm, zw_hbm,
                  accp_hbm, waccp_hbm, idxbuf, buf, wbuf, wout, acc, wacc):
    c = lax.axis_index("c")
    s = lax.axis_index("s")
    wid = s * 2 + c
    npw = K * CHUNK
    M = NW * npw
    base = wid * npw
    pltpu.sync_copy(idx_hbm.at[wid], idxbuf)

    for h in range(NHEAD):
        @pl.when(s == 0)
        def _zero():
            pltpu.sync_copy(zf_hbm, acc)
            pltpu.sync_copy(zw_hbm, wacc)

        plsc.subcore_barrier()

        def body(j, _):
            pltpu.sync_copy(wm_hbm.at[h, pl.ds(base + j * CHUNK, CHUNK), :], buf)
            pltpu.sync_copy(w_hbm.at[pl.ds(h * M + base + j * CHUNK, CHUNK)], wbuf)
            pltpu.sync_copy(buf, acc.at[idxbuf.at[j]], add=True)
            pltpu.sync_copy(wbuf, wacc.at[idxbuf.at[j]], add=True)
            return 0

        lax.fori_loop(0, K, body, 0, unroll=False)
        plsc.subcore_barrier()

        @pl.when(s == 0)
        def _flush():
            pltpu.sync_copy(acc, accp_hbm.at[c, h])
            pltpu.sync_copy(wacc, wout)
            pltpu.sync_copy(wout, waccp_hbm.at[pl.ds((c * NHEAD + h) * N, N)])

        if h < NHEAD - 1:
            plsc.subcore_barrier()


def _scatter(wm, w, idx3, N):
    K = idx3.shape[1]
    zf = jnp.zeros((N, FEA), jnp.float32)
    zw = jnp.zeros((N,), jnp.float32)
    f = pl.kernel(
        functools.partial(_scatter_body, K, N),
        out_type=[
            jax.ShapeDtypeStruct((2, NHEAD, N, FEA), jnp.float32),
            jax.ShapeDtypeStruct((2 * NHEAD * N,), jnp.float32),
        ],
        mesh=_sc_mesh(),
        scratch_types=[
            pltpu.VMEM((K, CHUNK), jnp.int32),
            pltpu.VMEM((CHUNK, FEA), jnp.float32),
            pltpu.VMEM((CHUNK,), jnp.float32),
            pltpu.VMEM((N,), jnp.float32),
            pltpu.VMEM_SHARED((N, FEA), jnp.float32),
            pltpu.VMEM_SHARED((N,), jnp.float32),
        ],
    )
    return f(wm, w.reshape(-1), idx3, zf, zw)


def _combine_kernel(accp_ref, waccp_ref, fea_ref, gamma_ref, beta_ref, out_ref):
    h = pl.program_id(0)
    num = accp_ref[0, 0] + accp_ref[1, 0]
    den = waccp_ref[0, 0] + waccp_ref[1, 0]
    contrib = num / (den + 1e-10)

    @pl.when(h == 0)
    def _init():
        out_ref[...] = contrib

    @pl.when(h > 0)
    def _acc():
        out_ref[...] += contrib

    @pl.when(h == NHEAD - 1)
    def _bn():
        fea = out_ref[...] * (1.0 / NHEAD) + fea_ref[...]
        m = jnp.mean(fea, axis=0, keepdims=True)
        v = jnp.mean((fea - m) ** 2, axis=0, keepdims=True)
        out_ref[...] = (fea - m) / jnp.sqrt(v + 1e-5) * gamma_ref[...] + beta_ref[...]


def _combine(accp, waccp, fea, gamma, beta):
    N = fea.shape[0]
    return pl.pallas_call(
        _combine_kernel,
        grid=(NHEAD,),
        in_specs=[
            pl.BlockSpec((2, 1, N, FEA), lambda h: (0, h, 0, 0)),
            pl.BlockSpec((2, 1, N, 1), lambda h: (0, h, 0, 0)),
            pl.BlockSpec((N, FEA), lambda h: (0, 0)),
            pl.BlockSpec((1, FEA), lambda h: (0, 0)),
            pl.BlockSpec((1, FEA), lambda h: (0, 0)),
        ],
        out_specs=pl.BlockSpec((N, FEA), lambda h: (0, 0)),
        out_shape=jax.ShapeDtypeStruct((N, FEA), jnp.float32),
    )(accp, waccp.reshape(2, NHEAD, N, 1), fea,
      gamma.reshape(1, -1), beta.reshape(1, -1))


def _cry_acc_kernel(nparams, fea_ref, ew_ref, cidx_ref, *rest):
    param_refs = rest[:nparams]
    acc_ref, wt_ref = rest[nparams:]
    i = pl.program_id(0)
    it = iter([r[...] for r in param_refs])

    def nxt():
        return next(it)

    fea = fea_ref[...]
    ew = ew_ref[...]
    lew = jnp.log(ew)
    cidx = cidx_ref[...]
    onehot = jnp.where(
        cidx == jax.lax.broadcasted_iota(jnp.int32, (NBLK, CRY), 1), 1.0, 0.0
    ).astype(jnp.float32)

    @pl.when(i == 0)
    def _init():
        acc_ref[...] = jnp.zeros_like(acc_ref)

    for h in range(NHEAD):
        Wg1, bg1, Wg2, bg2, Wm1, bm1, Wm2, bm2, pw = (nxt() for _ in range(9))
        logit = _dot(_leaky(_dot(fea, Wg1) + bg1), Wg2) + bg2
        w = jnp.exp(logit + pw[0, 0] * lew)
        msg = _dot(_leaky(_dot(fea, Wm1) + bm1), Wm2) + bm2
        vals = jnp.concatenate([w * msg, w], axis=1)
        acc_ref[h, :, :] += lax.dot_general(
            onehot, vals, (((0,), (0,)), ((), ())),
            preferred_element_type=jnp.float32)
        wt_ref[h, :, :] = w


def _cry_out_kernel(acc_ref, wt_ref, cidx_ref, crys_ref, gates_ref):
    i = pl.program_id(0)
    cidx = cidx_ref[...]
    onehot = jnp.where(
        cidx == jax.lax.broadcasted_iota(jnp.int32, (NBLK, CRY), 1), 1.0, 0.0
    ).astype(jnp.float32)

    @pl.when(i == 0)
    def _crys():
        tot = jnp.zeros((CRY, FEA), jnp.float32)
        for h in range(NHEAD):
            tot = tot + acc_ref[h, :, :FEA] / (acc_ref[h, :, FEA:] + 1e-10)
        crys_ref[...] = tot * (1.0 / NHEAD)

    for h in range(NHEAD):
        gsum = _dot(onehot, acc_ref[h, :, FEA:])
        gates_ref[h, :, :] = wt_ref[h, :, :] / (gsum + 1e-10)


def _flatten_cry_params(params):
    out = []
    for h in params["cry"]:
        (W1, b1), = h["gate"]["fcs"]
        W2, b2 = h["gate"]["out"]
        (V1, c1), = h["msg"]["fcs"]
        V2, c2 = h["msg"]["out"]
        out += [W1, b1.reshape(1, -1), W2, b2.reshape(1, -1),
                V1, c1.reshape(1, -1), V2, c2.reshape(1, -1),
                h["pow"].reshape(1, 1)]
    return out


def _crystal(fea, elem_weights, cry_idx_col, params):
    N = fea.shape[0]
    flat = _flatten_cry_params(params)
    nparams = len(flat)
    grid = (N // NBLK,)

    def full(a):
        return pl.BlockSpec(a.shape, lambda i: (0,) * a.ndim)

    acc, wt = pl.pallas_call(
        functools.partial(_cry_acc_kernel, nparams),
        grid=grid,
        in_specs=[
            pl.BlockSpec((NBLK, FEA), lambda i: (i, 0)),
            pl.BlockSpec((NBLK, 1), lambda i: (i, 0)),
            pl.BlockSpec((NBLK, 1), lambda i: (i, 0)),
        ] + [full(a) for a in flat],
        out_specs=[
            pl.BlockSpec((NHEAD, CRY, FEA + 1), lambda i: (0, 0, 0)),
            pl.BlockSpec((NHEAD, NBLK, 1), lambda i: (0, i, 0)),
        ],
        out_shape=[
            jax.ShapeDtypeStruct((NHEAD, CRY, FEA + 1), jnp.float32),
            jax.ShapeDtypeStruct((NHEAD, N, 1), jnp.float32),
        ],
    )(fea, elem_weights, cry_idx_col, *flat)

    crys, gates = pl.pallas_call(
        _cry_out_kernel,
        grid=grid,
        in_specs=[
            full(acc),
            pl.BlockSpec((NHEAD, NBLK, 1), lambda i: (0, i, 0)),
            pl.BlockSpec((NBLK, 1), lambda i: (i, 0)),
        ],
        out_specs=[
            pl.BlockSpec((CRY, FEA), lambda i: (0, 0)),
            pl.BlockSpec((NHEAD, NBLK, 1), lambda i: (0, i, 0)),
        ],
        out_shape=[
            jax.ShapeDtypeStruct((CRY, FEA), jnp.float32),
            jax.ShapeDtypeStruct((NHEAD, N, 1), jnp.float32),
        ],
    )(acc, wt, cry_idx_col)
    return crys, gates


def kernel(elem_weights, elem_fea, edge_fea, self_fea_idx, nbr_fea_idx, cry_elem_idx, params):
    N = elem_fea.shape[0]
    M = edge_fea.shape[0]
    K = M // (NW * CHUNK)

    sidx3 = self_fea_idx.astype(jnp.int32).reshape(NW, K, CHUNK)
    nidx3 = nbr_fea_idx.astype(jnp.int32).reshape(NW, K, CHUNK)

    W, b = params["embedding"]
    fea = _prep(elem_fea, elem_weights, W, b)
    nbr_w = None

    for g in params["graphs"]:
        self_f, nbr_f = _gather_rows(fea, sidx3, nidx3)
        flat = _flatten_layer_params(g)
        res = _edge_compute(self_f, nbr_f, edge_fea, nbr_w, flat)
        if nbr_w is None:
            w, wm, nbr_w = res
        else:
            w, wm = res
        accp, waccp = _scatter(wm, w, sidx3, N)
        gamma, beta = g["bn"]
        fea = _combine(accp, waccp, fea, gamma, beta)

    crys, gates = _crystal(fea, elem_weights,
                           cry_elem_idx.astype(jnp.int32).reshape(N, 1), params)
    return crys, gates


# speedup vs baseline: 8.2825x; 1.4746x over previous
import functools

import jax
import jax.numpy as jnp
from jax import lax
from jax.experimental import pallas as pl
from jax.experimental.pallas import tpu as pltpu
from jax.experimental.pallas import tpu_sc as plsc


FEA = 128
EDGE = 16
NHEAD = 3
NW = 32
CHUNK = 80
BLOCK_E = 2560
NBLK = 1000
CRY = 2000


def _leaky(x):
    return jnp.where(x >= 0, x, 0.01 * x)


def _dot(a, b):
    return jnp.dot(a, b, preferred_element_type=jnp.float32)


def _prep_kernel(ef_ref, ew_ref, w_ref, b_ref, out_ref):
    mm = _dot(ef_ref[...], w_ref[...]) + b_ref[...]
    out_ref[...] = jnp.concatenate([mm, ew_ref[...]], axis=1)


def _prep(elem_fea, elem_weights, W, b):
    N = elem_fea.shape[0]
    return pl.pallas_call(
        _prep_kernel,
        out_shape=jax.ShapeDtypeStruct((N, FEA), jnp.float32),
    )(elem_fea, elem_weights, W, b.reshape(1, -1))


def _sc_mesh():
    return plsc.VectorSubcoreMesh(core_axis_name="c", subcore_axis_name="s",
                                  num_cores=2, num_subcores=16)


def _gather_rows_body(K, fea_hbm, sidx_hbm, nidx_hbm, sout_hbm, nout_hbm,
                      idxbuf, rb0, rb1, sem0, sem1):
    c = lax.axis_index("c")
    s = lax.axis_index("s")
    wid = s * 2 + c
    npw = K * CHUNK
    base = wid * npw

    def run(idx_hbm, out_hbm):
        pltpu.sync_copy(idx_hbm.at[wid], idxbuf)

        def gath(j, rb, sem):
            return pltpu.async_copy(fea_hbm.at[idxbuf.at[j]], rb, sem)

        def wait(j, rb, sem):
            pltpu.make_async_copy(fea_hbm.at[idxbuf.at[j]], rb, sem).wait()

        def put(j, rb):
            pltpu.sync_copy(rb, out_hbm.at[pl.ds(base + j * CHUNK, CHUNK), :])

        gath(0, rb0, sem0)

        def body(t, _):
            j0 = 2 * t
            gath(j0 + 1, rb1, sem1)
            wait(j0, rb0, sem0)
            put(j0, rb0)

            @pl.when(j0 + 2 < K)
            def _nxt():
                gath(j0 + 2, rb0, sem0)

            wait(j0 + 1, rb1, sem1)
            put(j0 + 1, rb1)
            return 0

        lax.fori_loop(0, K // 2, body, 0, unroll=False)
        if K % 2 == 1:
            wait(K - 1, rb0, sem0)
            put(K - 1, rb0)

    run(sidx_hbm, sout_hbm)
    run(nidx_hbm, nout_hbm)


def _gather_rows(fea, sidx3, nidx3):
    K = sidx3.shape[1]
    M = NW * K * CHUNK
    f = pl.kernel(
        functools.partial(_gather_rows_body, K),
        out_type=[
            jax.ShapeDtypeStruct((M, FEA), jnp.float32),
            jax.ShapeDtypeStruct((M, FEA), jnp.float32),
        ],
        mesh=_sc_mesh(),
        scratch_types=[
            pltpu.VMEM((K, CHUNK), jnp.int32),
            pltpu.VMEM((CHUNK, FEA), jnp.float32),
            pltpu.VMEM((CHUNK, FEA), jnp.float32),
            pltpu.SemaphoreType.DMA,
            pltpu.SemaphoreType.DMA,
        ],
    )
    return f(fea, sidx3, nidx3)


def _edge_block_kernel(nparams, first, self_f_ref, nbr_f_ref, ef_ref, *rest):
    if first:
        param_refs = rest[:nparams]
        w_ref, wm_ref, nw_ref = rest[nparams:]
    else:
        nw_in = rest[0]
        param_refs = rest[1:1 + nparams]
        w_ref, wm_ref = rest[1 + nparams:]
    it = iter([r[...] for r in param_refs])

    def nxt():
        return next(it)

    self_f = self_f_ref[...]
    nbr_f = nbr_f_ref[...]
    if first:
        nbr_w = nbr_f[:, FEA - 1:FEA]
        nw_ref[...] = nbr_w
    else:
        nbr_w = nw_in[...]
    lw = jnp.log(nbr_w)
    x = ef_ref[...]
    for _ in range(4):
        W, b = nxt(), nxt()
        x = _leaky(_dot(x, W) + b)
    W, b = nxt(), nxt()
    ef = _dot(x, W) + b

    zs = []
    msgs = []
    for h in range(NHEAD):
        Wg_s, Wg_n, Wg_e, bg1, Wg2, bg2 = (nxt() for _ in range(6))
        Wm_s, Wm_n, Wm_e, bm1, Wm2, bm2 = (nxt() for _ in range(6))
        pw = nxt()
        gh = _leaky(_dot(self_f, Wg_s) + _dot(nbr_f, Wg_n) + _dot(ef, Wg_e) + bg1)
        logit = _dot(gh, Wg2) + bg2
        zs.append(logit + pw[0, 0] * lw)
        mh = _leaky(_dot(self_f, Wm_s) + _dot(nbr_f, Wm_n) + _dot(ef, Wm_e) + bm1)
        msgs.append(_dot(mh, Wm2) + bm2)
    wall = jnp.exp(jnp.concatenate(zs, axis=1))
    for h in range(NHEAD):
        w = wall[:, h:h + 1]
        w_ref[h, :] = w[:, 0]
        wm_ref[h, :, :] = w * msgs[h]


def _flatten_layer_params(g):
    out = []
    for W, b in g["edge_ebd"]["fcs"]:
        out += [W, b.reshape(1, -1)]
    W, b = g["edge_ebd"]["out"]
    out += [W, b.reshape(1, -1)]
    for h in g["heads"]:
        for mlp in (h["gate"], h["msg"]):
            (W1, b1), = mlp["fcs"]
            W2, b2 = mlp["out"]
            out += [W1[:FEA], W1[FEA:2 * FEA], W1[2 * FEA:], b1.reshape(1, -1),
                    W2, b2.reshape(1, -1)]
        out += [h["pow"].reshape(1, 1)]
    return out


def _edge_compute(self_f, nbr_f, edge_fea, nbr_w, flat_params):
    M = edge_fea.shape[0]
    grid = (M // BLOCK_E,)
    nparams = len(flat_params)
    first = nbr_w is None

    def full(a):
        return pl.BlockSpec(a.shape, lambda i: (0,) * a.ndim)

    in_specs = [
        pl.BlockSpec((BLOCK_E, FEA), lambda i: (i, 0)),
        pl.BlockSpec((BLOCK_E, FEA), lambda i: (i, 0)),
        pl.BlockSpec((BLOCK_E, EDGE), lambda i: (i, 0)),
    ]
    operands = [self_f, nbr_f, edge_fea]
    if not first:
        in_specs.append(pl.BlockSpec((BLOCK_E, 1), lambda i: (i, 0)))
        operands.append(nbr_w)
    in_specs += [full(a) for a in flat_params]
    operands += flat_params
    out_specs = [
        pl.BlockSpec((NHEAD, BLOCK_E), lambda i: (0, i)),
        pl.BlockSpec((NHEAD, BLOCK_E, FEA), lambda i: (0, i, 0)),
    ]
    out_shapes = [
        jax.ShapeDtypeStruct((NHEAD, M), jnp.float32),
        jax.ShapeDtypeStruct((NHEAD, M, FEA), jnp.float32),
    ]
    if first:
        out_specs.append(pl.BlockSpec((BLOCK_E, 1), lambda i: (i, 0)))
        out_shapes.append(jax.ShapeDtypeStruct((M, 1), jnp.float32))
    return pl.pallas_call(
        functools.partial(_edge_block_kernel, nparams, first),
        grid=grid,
        in_specs=in_specs,
        out_specs=out_specs,
        out_shape=out_shapes,
    )(*operands)


def _scatter_body(K, N, wm_hbm, w_hbm, idx_hbm, zf_hbm, zw_hbm,
                  accp_hbm, waccp_hbm, idxbuf, buf0, buf1, wbuf0, wbuf1,
                  wout, sem0, sem1, acc, wacc):
    c = lax.axis_index("c")
    s = lax.axis_index("s")
    wid = s * 2 + c
    npw = K * CHUNK
    M = NW * npw
    base = wid * npw
    pltpu.sync_copy(idx_hbm.at[wid], idxbuf)

    for h in range(NHEAD):
        @pl.when(s == 0)
        def _zero():
            pltpu.sync_copy(zf_hbm, acc)
            pltpu.sync_copy(zw_hbm, wacc)

        plsc.subcore_barrier()

        def load(j, b, wb, sem):
            pltpu.async_copy(wm_hbm.at[h, pl.ds(base + j * CHUNK, CHUNK), :], b, sem)
            pltpu.async_copy(w_hbm.at[pl.ds(h * M + base + j * CHUNK, CHUNK)], wb, sem)

        def wait(j, b, wb, sem):
            pltpu.make_async_copy(wm_hbm.at[h, pl.ds(base + j * CHUNK, CHUNK), :], b, sem).wait()
            pltpu.make_async_copy(w_hbm.at[pl.ds(h * M + base + j * CHUNK, CHUNK)], wb, sem).wait()

        def scat(j, b, wb):
            pltpu.sync_copy(b, acc.at[idxbuf.at[j]], add=True)
            pltpu.sync_copy(wb, wacc.at[idxbuf.at[j]], add=True)

        load(0, buf0, wbuf0, sem0)

        def body(t, _):
            j0 = 2 * t
            load(j0 + 1, buf1, wbuf1, sem1)
            wait(j0, buf0, wbuf0, sem0)
            scat(j0, buf0, wbuf0)

            @pl.when(j0 + 2 < K)
            def _nxt():
                load(j0 + 2, buf0, wbuf0, sem0)

            wait(j0 + 1, buf1, wbuf1, sem1)
            scat(j0 + 1, buf1, wbuf1)
            return 0

        lax.fori_loop(0, K // 2, body, 0, unroll=False)
        if K % 2 == 1:
            wait(K - 1, buf0, wbuf0, sem0)
            scat(K - 1, buf0, wbuf0)
        plsc.subcore_barrier()

        @pl.when(s == 0)
        def _flush():
            pltpu.sync_copy(acc, accp_hbm.at[c, h])
            pltpu.sync_copy(wacc, wout)
            pltpu.sync_copy(wout, waccp_hbm.at[pl.ds((c * NHEAD + h) * N, N)])

        if h < NHEAD - 1:
            plsc.subcore_barrier()


def _scatter(wm, w, idx3, N):
    K = idx3.shape[1]
    zf = jnp.zeros((N, FEA), jnp.float32)
    zw = jnp.zeros((N,), jnp.float32)
    f = pl.kernel(
        functools.partial(_scatter_body, K, N),
        out_type=[
            jax.ShapeDtypeStruct((2, NHEAD, N, FEA), jnp.float32),
            jax.ShapeDtypeStruct((2 * NHEAD * N,), jnp.float32),
        ],
        mesh=_sc_mesh(),
        scratch_types=[
            pltpu.VMEM((K, CHUNK), jnp.int32),
            pltpu.VMEM((CHUNK, FEA), jnp.float32),
            pltpu.VMEM((CHUNK, FEA), jnp.float32),
            pltpu.VMEM((CHUNK,), jnp.float32),
            pltpu.VMEM((CHUNK,), jnp.float32),
            pltpu.VMEM((N,), jnp.float32),
            pltpu.SemaphoreType.DMA,
            pltpu.SemaphoreType.DMA,
            pltpu.VMEM_SHARED((N, FEA), jnp.float32),
            pltpu.VMEM_SHARED((N,), jnp.float32),
        ],
    )
    return f(wm, w.reshape(-1), idx3, zf, zw)


def _combine_kernel(accp_ref, waccp_ref, fea_ref, gamma_ref, beta_ref, out_ref):
    h = pl.program_id(0)
    num = accp_ref[0, 0] + accp_ref[1, 0]
    den = waccp_ref[0, 0] + waccp_ref[1, 0]
    contrib = num / (den + 1e-10)

    @pl.when(h == 0)
    def _init():
        out_ref[...] = contrib

    @pl.when(h > 0)
    def _acc():
        out_ref[...] += contrib

    @pl.when(h == NHEAD - 1)
    def _bn():
        fea = out_ref[...] * (1.0 / NHEAD) + fea_ref[...]
        m = jnp.mean(fea, axis=0, keepdims=True)
        v = jnp.mean((fea - m) ** 2, axis=0, keepdims=True)
        out_ref[...] = (fea - m) / jnp.sqrt(v + 1e-5) * gamma_ref[...] + beta_ref[...]


def _combine(accp, waccp, fea, gamma, beta):
    N = fea.shape[0]
    return pl.pallas_call(
        _combine_kernel,
        grid=(NHEAD,),
        in_specs=[
            pl.BlockSpec((2, 1, N, FEA), lambda h: (0, h, 0, 0)),
            pl.BlockSpec((2, 1, N, 1), lambda h: (0, h, 0, 0)),
            pl.BlockSpec((N, FEA), lambda h: (0, 0)),
            pl.BlockSpec((1, FEA), lambda h: (0, 0)),
            pl.BlockSpec((1, FEA), lambda h: (0, 0)),
        ],
        out_specs=pl.BlockSpec((N, FEA), lambda h: (0, 0)),
        out_shape=jax.ShapeDtypeStruct((N, FEA), jnp.float32),
    )(accp, waccp.reshape(2, NHEAD, N, 1), fea,
      gamma.reshape(1, -1), beta.reshape(1, -1))


def _cry_acc_kernel(nparams, fea_ref, ew_ref, cidx_ref, *rest):
    param_refs = rest[:nparams]
    acc_ref, wt_ref = rest[nparams:]
    i = pl.program_id(0)
    it = iter([r[...] for r in param_refs])

    def nxt():
        return next(it)

    fea = fea_ref[...]
    ew = ew_ref[...]
    lew = jnp.log(ew)
    cidx = cidx_ref[...]
    onehot = jnp.where(
        cidx == jax.lax.broadcasted_iota(jnp.int32, (NBLK, CRY), 1), 1.0, 0.0
    ).astype(jnp.float32)

    @pl.when(i == 0)
    def _init():
        acc_ref[...] = jnp.zeros_like(acc_ref)

    for h in range(NHEAD):
        Wg1, bg1, Wg2, bg2, Wm1, bm1, Wm2, bm2, pw = (nxt() for _ in range(9))
        logit = _dot(_leaky(_dot(fea, Wg1) + bg1), Wg2) + bg2
        w = jnp.exp(logit + pw[0, 0] * lew)
        msg = _dot(_leaky(_dot(fea, Wm1) + bm1), Wm2) + bm2
        vals = jnp.concatenate([w * msg, w], axis=1)
        acc_ref[h, :, :] += lax.dot_general(
            onehot, vals, (((0,), (0,)), ((), ())),
            preferred_element_type=jnp.float32)
        wt_ref[h, :, :] = w


def _cry_out_kernel(acc_ref, wt_ref, cidx_ref, crys_ref, gates_ref):
    i = pl.program_id(0)
    cidx = cidx_ref[...]
    onehot = jnp.where(
        cidx == jax.lax.broadcasted_iota(jnp.int32, (NBLK, CRY), 1), 1.0, 0.0
    ).astype(jnp.float32)

    @pl.when(i == 0)
    def _crys():
        tot = jnp.zeros((CRY, FEA), jnp.float32)
        for h in range(NHEAD):
            tot = tot + acc_ref[h, :, :FEA] / (acc_ref[h, :, FEA:] + 1e-10)
        crys_ref[...] = tot * (1.0 / NHEAD)

    for h in range(NHEAD):
        gsum = _dot(onehot, acc_ref[h, :, FEA:])
        gates_ref[h, :, :] = wt_ref[h, :, :] / (gsum + 1e-10)


def _flatten_cry_params(params):
    out = []
    for h in params["cry"]:
        (W1, b1), = h["gate"]["fcs"]
        W2, b2 = h["gate"]["out"]
        (V1, c1), = h["msg"]["fcs"]
        V2, c2 = h["msg"]["out"]
        out += [W1, b1.reshape(1, -1), W2, b2.reshape(1, -1),
                V1, c1.reshape(1, -1), V2, c2.reshape(1, -1),
                h["pow"].reshape(1, 1)]
    return out


def _crystal(fea, elem_weights, cry_idx_col, params):
    N = fea.shape[0]
    flat = _flatten_cry_params(params)
    nparams = len(flat)
    grid = (N // NBLK,)

    def full(a):
        return pl.BlockSpec(a.shape, lambda i: (0,) * a.ndim)

    acc, wt = pl.pallas_call(
        functools.partial(_cry_acc_kernel, nparams),
        grid=grid,
        in_specs=[
            pl.BlockSpec((NBLK, FEA), lambda i: (i, 0)),
            pl.BlockSpec((NBLK, 1), lambda i: (i, 0)),
            pl.BlockSpec((NBLK, 1), lambda i: (i, 0)),
        ] + [full(a) for a in flat],
        out_specs=[
            pl.BlockSpec((NHEAD, CRY, FEA + 1), lambda i: (0, 0, 0)),
            pl.BlockSpec((NHEAD, NBLK, 1), lambda i: (0, i, 0)),
        ],
        out_shape=[
            jax.ShapeDtypeStruct((NHEAD, CRY, FEA + 1), jnp.float32),
            jax.ShapeDtypeStruct((NHEAD, N, 1), jnp.float32),
        ],
    )(fea, elem_weights, cry_idx_col, *flat)

    crys, gates = pl.pallas_call(
        _cry_out_kernel,
        grid=grid,
        in_specs=[
            full(acc),
            pl.BlockSpec((NHEAD, NBLK, 1), lambda i: (0, i, 0)),
            pl.BlockSpec((NBLK, 1), lambda i: (i, 0)),
        ],
        out_specs=[
            pl.BlockSpec((CRY, FEA), lambda i: (0, 0)),
            pl.BlockSpec((NHEAD, NBLK, 1), lambda i: (0, i, 0)),
        ],
        out_shape=[
            jax.ShapeDtypeStruct((CRY, FEA), jnp.float32),
            jax.ShapeDtypeStruct((NHEAD, N, 1), jnp.float32),
        ],
    )(acc, wt, cry_idx_col)
    return crys, gates


def kernel(elem_weights, elem_fea, edge_fea, self_fea_idx, nbr_fea_idx, cry_elem_idx, params):
    N = elem_fea.shape[0]
    M = edge_fea.shape[0]
    K = M // (NW * CHUNK)

    sidx3 = self_fea_idx.astype(jnp.int32).reshape(NW, K, CHUNK)
    nidx3 = nbr_fea_idx.astype(jnp.int32).reshape(NW, K, CHUNK)

    W, b = params["embedding"]
    fea = _prep(elem_fea, elem_weights, W, b)
    nbr_w = None

    for g in params["graphs"]:
        self_f, nbr_f = _gather_rows(fea, sidx3, nidx3)
        flat = _flatten_layer_params(g)
        res = _edge_compute(self_f, nbr_f, edge_fea, nbr_w, flat)
        if nbr_w is None:
            w, wm, nbr_w = res
        else:
            w, wm = res
        accp, waccp = _scatter(wm, w, sidx3, N)
        gamma, beta = g["bn"]
        fea = _combine(accp, waccp, fea, gamma, beta)

    crys, gates = _crystal(fea, elem_weights,
                           cry_elem_idx.astype(jnp.int32).reshape(N, 1), params)
    return crys, gates


# speedup vs baseline: 8.7763x; 1.0596x over previous
import functools

import jax
import jax.numpy as jnp
from jax import lax
from jax.experimental import pallas as pl
from jax.experimental.pallas import tpu as pltpu
from jax.experimental.pallas import tpu_sc as plsc


FEA = 128
EDGE = 16
NHEAD = 3
NW = 32
CHUNK = 80
BLOCK_E = 2560
NBLK = 1000
CRY = 2000


def _leaky(x):
    return jnp.where(x >= 0, x, 0.01 * x)


def _dot(a, b):
    return jnp.dot(a, b, preferred_element_type=jnp.float32)


def _prep_kernel(ef_ref, ew_ref, w_ref, b_ref, out_ref):
    mm = _dot(ef_ref[...], w_ref[...]) + b_ref[...]
    out_ref[...] = jnp.concatenate([mm, ew_ref[...]], axis=1)


def _prep(elem_fea, elem_weights, W, b):
    N = elem_fea.shape[0]
    return pl.pallas_call(
        _prep_kernel,
        out_shape=jax.ShapeDtypeStruct((N, FEA), jnp.float32),
    )(elem_fea, elem_weights, W, b.reshape(1, -1))


def _sc_mesh():
    return plsc.VectorSubcoreMesh(core_axis_name="c", subcore_axis_name="s",
                                  num_cores=2, num_subcores=16)


def _gather_rows_body(K, fea_hbm, sidx_hbm, nidx_hbm, sout_hbm, nout_hbm,
                      idxbuf, rb0, rb1, sem0, sem1):
    c = lax.axis_index("c")
    s = lax.axis_index("s")
    wid = s * 2 + c
    npw = K * CHUNK
    base = wid * npw

    def run(idx_hbm, out_hbm):
        pltpu.sync_copy(idx_hbm.at[wid], idxbuf)

        def gath(j, rb, sem):
            return pltpu.async_copy(fea_hbm.at[idxbuf.at[j]], rb, sem)

        def wait(j, rb, sem):
            pltpu.make_async_copy(fea_hbm.at[idxbuf.at[j]], rb, sem).wait()

        def put(j, rb):
            pltpu.sync_copy(rb, out_hbm.at[pl.ds(base + j * CHUNK, CHUNK), :])

        gath(0, rb0, sem0)

        def body(t, _):
            j0 = 2 * t
            gath(j0 + 1, rb1, sem1)
            wait(j0, rb0, sem0)
            put(j0, rb0)

            @pl.when(j0 + 2 < K)
            def _nxt():
                gath(j0 + 2, rb0, sem0)

            wait(j0 + 1, rb1, sem1)
            put(j0 + 1, rb1)
            return 0

        lax.fori_loop(0, K // 2, body, 0, unroll=False)
        if K % 2 == 1:
            wait(K - 1, rb0, sem0)
            put(K - 1, rb0)

    run(sidx_hbm, sout_hbm)
    run(nidx_hbm, nout_hbm)


def _gather_rows(fea, sidx3, nidx3):
    K = sidx3.shape[1]
    M = NW * K * CHUNK
    f = pl.kernel(
        functools.partial(_gather_rows_body, K),
        out_type=[
            jax.ShapeDtypeStruct((M, FEA), jnp.float32),
            jax.ShapeDtypeStruct((M, FEA), jnp.float32),
        ],
        mesh=_sc_mesh(),
        scratch_types=[
            pltpu.VMEM((K, CHUNK), jnp.int32),
            pltpu.VMEM((CHUNK, FEA), jnp.float32),
            pltpu.VMEM((CHUNK, FEA), jnp.float32),
            pltpu.SemaphoreType.DMA,
            pltpu.SemaphoreType.DMA,
        ],
    )
    return f(fea, sidx3, nidx3)


def _edge_block_kernel(nparams, first, self_f_ref, nbr_f_ref, ef_ref, *rest):
    if first:
        param_refs = rest[:nparams]
        w_ref, wm_ref, nw_ref = rest[nparams:]
    else:
        nw_in = rest[0]
        param_refs = rest[1:1 + nparams]
        w_ref, wm_ref = rest[1 + nparams:]
    it = iter([r[...] for r in param_refs])

    def nxt():
        return next(it)

    self_f = self_f_ref[...]
    nbr_f = nbr_f_ref[...]
    if first:
        nbr_w = nbr_f[:, FEA - 1:FEA]
        nw_ref[...] = nbr_w
    else:
        nbr_w = nw_in[...]
    lw = jnp.log(nbr_w)
    x = ef_ref[...]
    for _ in range(4):
        W, b = nxt(), nxt()
        x = _leaky(_dot(x, W) + b)
    W, b = nxt(), nxt()
    ef = _dot(x, W) + b

    zs = []
    msgs = []
    for h in range(NHEAD):
        Wg_s, Wg_n, Wg_e, bg1, Wg2, bg2 = (nxt() for _ in range(6))
        Wm_s, Wm_n, Wm_e, bm1, Wm2, bm2 = (nxt() for _ in range(6))
        pw = nxt()
        gh = _leaky(_dot(self_f, Wg_s) + _dot(nbr_f, Wg_n) + _dot(ef, Wg_e) + bg1)
        logit = _dot(gh, Wg2) + bg2
        zs.append(logit + pw[0, 0] * lw)
        mh = _leaky(_dot(self_f, Wm_s) + _dot(nbr_f, Wm_n) + _dot(ef, Wm_e) + bm1)
        msgs.append(_dot(mh, Wm2) + bm2)
    wall = jnp.exp(jnp.concatenate(zs, axis=1))
    for h in range(NHEAD):
        w = wall[:, h:h + 1]
        w_ref[h, :] = w[:, 0]
        wm_ref[h, :, :] = w * msgs[h]


def _flatten_layer_params(g):
    out = []
    for W, b in g["edge_ebd"]["fcs"]:
        out += [W, b.reshape(1, -1)]
    W, b = g["edge_ebd"]["out"]
    out += [W, b.reshape(1, -1)]
    for h in g["heads"]:
        for mlp in (h["gate"], h["msg"]):
            (W1, b1), = mlp["fcs"]
            W2, b2 = mlp["out"]
            out += [W1[:FEA], W1[FEA:2 * FEA], W1[2 * FEA:], b1.reshape(1, -1),
                    W2, b2.reshape(1, -1)]
        out += [h["pow"].reshape(1, 1)]
    return out


def _edge_compute(self_f, nbr_f, edge_fea, nbr_w, flat_params):
    M = edge_fea.shape[0]
    grid = (M // BLOCK_E,)
    nparams = len(flat_params)
    first = nbr_w is None

    def full(a):
        return pl.BlockSpec(a.shape, lambda i: (0,) * a.ndim)

    in_specs = [
        pl.BlockSpec((BLOCK_E, FEA), lambda i: (i, 0)),
        pl.BlockSpec((BLOCK_E, FEA), lambda i: (i, 0)),
        pl.BlockSpec((BLOCK_E, EDGE), lambda i: (i, 0)),
    ]
    operands = [self_f, nbr_f, edge_fea]
    if not first:
        in_specs.append(pl.BlockSpec((BLOCK_E, 1), lambda i: (i, 0)))
        operands.append(nbr_w)
    in_specs += [full(a) for a in flat_params]
    operands += flat_params
    out_specs = [
        pl.BlockSpec((NHEAD, BLOCK_E), lambda i: (0, i)),
        pl.BlockSpec((NHEAD, BLOCK_E, FEA), lambda i: (0, i, 0)),
    ]
    out_shapes = [
        jax.ShapeDtypeStruct((NHEAD, M), jnp.float32),
        jax.ShapeDtypeStruct((NHEAD, M, FEA), jnp.float32),
    ]
    if first:
        out_specs.append(pl.BlockSpec((BLOCK_E, 1), lambda i: (i, 0)))
        out_shapes.append(jax.ShapeDtypeStruct((M, 1), jnp.float32))
    return pl.pallas_call(
        functools.partial(_edge_block_kernel, nparams, first),
        grid=grid,
        in_specs=in_specs,
        out_specs=out_specs,
        out_shape=out_shapes,
    )(*operands)


def _scatter_body(K, N, wm_hbm, w_hbm, idx_hbm, zf_hbm, zw_hbm,
                  accp_hbm, waccp_hbm, idxbuf, buf0, buf1, wbuf0, wbuf1,
                  wout, sem0, sem1, acc, wacc):
    c = lax.axis_index("c")
    s = lax.axis_index("s")
    wid = s * 2 + c
    npw = K * CHUNK
    M = NW * npw
    base = wid * npw
    pltpu.sync_copy(idx_hbm.at[wid], idxbuf)

    for h in range(NHEAD):
        @pl.when(s == 0)
        def _zero():
            pltpu.sync_copy(zf_hbm, acc)
            pltpu.sync_copy(zw_hbm, wacc)

        plsc.subcore_barrier()

        def load(j, b, wb, sem):
            pltpu.async_copy(wm_hbm.at[h, pl.ds(base + j * CHUNK, CHUNK), :], b, sem)
            pltpu.async_copy(w_hbm.at[pl.ds(h * M + base + j * CHUNK, CHUNK)], wb, sem)

        def wait(j, b, wb, sem):
            pltpu.make_async_copy(wm_hbm.at[h, pl.ds(base + j * CHUNK, CHUNK), :], b, sem).wait()
            pltpu.make_async_copy(w_hbm.at[pl.ds(h * M + base + j * CHUNK, CHUNK)], wb, sem).wait()

        def scat(j, b, wb):
            pltpu.sync_copy(b, acc.at[idxbuf.at[j]], add=True)
            pltpu.sync_copy(wb, wacc.at[idxbuf.at[j]], add=True)

        load(0, buf0, wbuf0, sem0)

        def body(t, _):
            j0 = 2 * t
            load(j0 + 1, buf1, wbuf1, sem1)
            wait(j0, buf0, wbuf0, sem0)
            scat(j0, buf0, wbuf0)

            @pl.when(j0 + 2 < K)
            def _nxt():
                load(j0 + 2, buf0, wbuf0, sem0)

            wait(j0 + 1, buf1, wbuf1, sem1)
            scat(j0 + 1, buf1, wbuf1)
            return 0

        lax.fori_loop(0, K // 2, body, 0, unroll=False)
        if K % 2 == 1:
            wait(K - 1, buf0, wbuf0, sem0)
            scat(K - 1, buf0, wbuf0)
        plsc.subcore_barrier()

        @pl.when(s == 0)
        def _flush():
            pltpu.sync_copy(acc, accp_hbm.at[c, h])
            pltpu.sync_copy(wacc, wout)
            pltpu.sync_copy(wout, waccp_hbm.at[pl.ds((c * NHEAD + h) * N, N)])

        if h < NHEAD - 1:
            plsc.subcore_barrier()


def _scatter(wm, w, idx3, N):
    K = idx3.shape[1]
    zf = jnp.zeros((N, FEA), jnp.float32)
    zw = jnp.zeros((N,), jnp.float32)
    f = pl.kernel(
        functools.partial(_scatter_body, K, N),
        out_type=[
            jax.ShapeDtypeStruct((2, NHEAD, N, FEA), jnp.float32),
            jax.ShapeDtypeStruct((2 * NHEAD * N,), jnp.float32),
        ],
        mesh=_sc_mesh(),
        scratch_types=[
            pltpu.VMEM((K, CHUNK), jnp.int32),
            pltpu.VMEM((CHUNK, FEA), jnp.float32),
            pltpu.VMEM((CHUNK, FEA), jnp.float32),
            pltpu.VMEM((CHUNK,), jnp.float32),
            pltpu.VMEM((CHUNK,), jnp.float32),
            pltpu.VMEM((N,), jnp.float32),
            pltpu.SemaphoreType.DMA,
            pltpu.SemaphoreType.DMA,
            pltpu.VMEM_SHARED((N, FEA), jnp.float32),
            pltpu.VMEM_SHARED((N,), jnp.float32),
        ],
    )
    return f(wm, w.reshape(-1), idx3, zf, zw)


def _den_kernel(w_ref, inv_ref):
    den = w_ref[0, 0] + w_ref[1, 0] + w_ref[2, 0] + w_ref[3, 0]
    inv_ref[0] = 1.0 / (den + 1e-10)


def _combine_kernel(acc_ref, inv_ref, fea_ref, gamma_ref, beta_ref, out_ref):
    h = pl.program_id(0)
    p = pl.program_id(1)
    contrib = acc_ref[0, 0] * inv_ref[0]

    @pl.when((h == 0) & (p == 0))
    def _init():
        out_ref[...] = contrib

    @pl.when((h > 0) | (p > 0))
    def _acc():
        out_ref[...] += contrib

    @pl.when((h == NHEAD - 1) & (p == 3))
    def _bn():
        fea = out_ref[...] * (1.0 / NHEAD) + fea_ref[...]
        m = jnp.mean(fea, axis=0, keepdims=True)
        v = jnp.mean((fea - m) ** 2, axis=0, keepdims=True)
        out_ref[...] = (fea - m) / jnp.sqrt(v + 1e-5) * gamma_ref[...] + beta_ref[...]


def _combine(accp0, accp1, waccp0, waccp1, fea, gamma, beta):
    N = fea.shape[0]
    accp = jnp.concatenate([accp0, accp1], axis=0)
    waccp = jnp.concatenate([waccp0.reshape(2, NHEAD, N, 1),
                             waccp1.reshape(2, NHEAD, N, 1)], axis=0)
    inv = pl.pallas_call(
        _den_kernel,
        grid=(NHEAD,),
        in_specs=[pl.BlockSpec((4, 1, N, 1), lambda h: (0, h, 0, 0))],
        out_specs=pl.BlockSpec((1, N, 1), lambda h: (h, 0, 0)),
        out_shape=jax.ShapeDtypeStruct((NHEAD, N, 1), jnp.float32),
    )(waccp)
    return pl.pallas_call(
        _combine_kernel,
        grid=(NHEAD, 4),
        in_specs=[
            pl.BlockSpec((1, 1, N, FEA), lambda h, p: (p, h, 0, 0)),
            pl.BlockSpec((1, N, 1), lambda h, p: (h, 0, 0)),
            pl.BlockSpec((N, FEA), lambda h, p: (0, 0)),
            pl.BlockSpec((1, FEA), lambda h, p: (0, 0)),
            pl.BlockSpec((1, FEA), lambda h, p: (0, 0)),
        ],
        out_specs=pl.BlockSpec((N, FEA), lambda h, p: (0, 0)),
        out_shape=jax.ShapeDtypeStruct((N, FEA), jnp.float32),
    )(accp, inv, fea, gamma.reshape(1, -1), beta.reshape(1, -1))


def _cry_acc_kernel(nparams, fea_ref, ew_ref, cidx_ref, *rest):
    param_refs = rest[:nparams]
    acc_ref, wt_ref = rest[nparams:]
    i = pl.program_id(0)
    it = iter([r[...] for r in param_refs])

    def nxt():
        return next(it)

    fea = fea_ref[...]
    ew = ew_ref[...]
    lew = jnp.log(ew)
    cidx = cidx_ref[...]
    onehot = jnp.where(
        cidx == jax.lax.broadcasted_iota(jnp.int32, (NBLK, CRY), 1), 1.0, 0.0
    ).astype(jnp.float32)

    @pl.when(i == 0)
    def _init():
        acc_ref[...] = jnp.zeros_like(acc_ref)

    for h in range(NHEAD):
        Wg1, bg1, Wg2, bg2, Wm1, bm1, Wm2, bm2, pw = (nxt() for _ in range(9))
        logit = _dot(_leaky(_dot(fea, Wg1) + bg1), Wg2) + bg2
        w = jnp.exp(logit + pw[0, 0] * lew)
        msg = _dot(_leaky(_dot(fea, Wm1) + bm1), Wm2) + bm2
        vals = jnp.concatenate([w * msg, w], axis=1)
        acc_ref[h, :, :] += lax.dot_general(
            onehot, vals, (((0,), (0,)), ((), ())),
            preferred_element_type=jnp.float32)
        wt_ref[h, :, :] = w


def _cry_out_kernel(acc_ref, wt_ref, cidx_ref, crys_ref, gates_ref):
    i = pl.program_id(0)
    cidx = cidx_ref[...]
    onehot = jnp.where(
        cidx == jax.lax.broadcasted_iota(jnp.int32, (NBLK, CRY), 1), 1.0, 0.0
    ).astype(jnp.float32)

    @pl.when(i == 0)
    def _crys():
        tot = jnp.zeros((CRY, FEA), jnp.float32)
        for h in range(NHEAD):
            tot = tot + acc_ref[h, :, :FEA] / (acc_ref[h, :, FEA:] + 1e-10)
        crys_ref[...] = tot * (1.0 / NHEAD)

    for h in range(NHEAD):
        gsum = _dot(onehot, acc_ref[h, :, FEA:])
        gates_ref[h, :, :] = wt_ref[h, :, :] / (gsum + 1e-10)


def _flatten_cry_params(params):
    out = []
    for h in params["cry"]:
        (W1, b1), = h["gate"]["fcs"]
        W2, b2 = h["gate"]["out"]
        (V1, c1), = h["msg"]["fcs"]
        V2, c2 = h["msg"]["out"]
        out += [W1, b1.reshape(1, -1), W2, b2.reshape(1, -1),
                V1, c1.reshape(1, -1), V2, c2.reshape(1, -1),
                h["pow"].reshape(1, 1)]
    return out


def _crystal(fea, elem_weights, cry_idx_col, params):
    N = fea.shape[0]
    flat = _flatten_cry_params(params)
    nparams = len(flat)
    grid = (N // NBLK,)

    def full(a):
        return pl.BlockSpec(a.shape, lambda i: (0,) * a.ndim)

    acc, wt = pl.pallas_call(
        functools.partial(_cry_acc_kernel, nparams),
        grid=grid,
        in_specs=[
            pl.BlockSpec((NBLK, FEA), lambda i: (i, 0)),
            pl.BlockSpec((NBLK, 1), lambda i: (i, 0)),
            pl.BlockSpec((NBLK, 1), lambda i: (i, 0)),
        ] + [full(a) for a in flat],
        out_specs=[
            pl.BlockSpec((NHEAD, CRY, FEA + 1), lambda i: (0, 0, 0)),
            pl.BlockSpec((NHEAD, NBLK, 1), lambda i: (0, i, 0)),
        ],
        out_shape=[
            jax.ShapeDtypeStruct((NHEAD, CRY, FEA + 1), jnp.float32),
            jax.ShapeDtypeStruct((NHEAD, N, 1), jnp.float32),
        ],
    )(fea, elem_weights, cry_idx_col, *flat)

    crys, gates = pl.pallas_call(
        _cry_out_kernel,
        grid=grid,
        in_specs=[
            full(acc),
            pl.BlockSpec((NHEAD, NBLK, 1), lambda i: (0, i, 0)),
            pl.BlockSpec((NBLK, 1), lambda i: (i, 0)),
        ],
        out_specs=[
            pl.BlockSpec((CRY, FEA), lambda i: (0, 0)),
            pl.BlockSpec((NHEAD, NBLK, 1), lambda i: (0, i, 0)),
        ],
        out_shape=[
            jax.ShapeDtypeStruct((CRY, FEA), jnp.float32),
            jax.ShapeDtypeStruct((NHEAD, N, 1), jnp.float32),
        ],
    )(acc, wt, cry_idx_col)
    return crys, gates


def kernel(elem_weights, elem_fea, edge_fea, self_fea_idx, nbr_fea_idx, cry_elem_idx, params):
    N = elem_fea.shape[0]
    M = edge_fea.shape[0]
    K = M // (NW * CHUNK)
    K0 = (K + 1) // 2
    M0 = NW * K0 * CHUNK
    sidx = self_fea_idx.astype(jnp.int32)
    nidx = nbr_fea_idx.astype(jnp.int32)
    parts = [
        (sidx[:M0].reshape(NW, K0, CHUNK), nidx[:M0].reshape(NW, K0, CHUNK),
         edge_fea[:M0]),
        (sidx[M0:].reshape(NW, K - K0, CHUNK), nidx[M0:].reshape(NW, K - K0, CHUNK),
         edge_fea[M0:]),
    ]

    W, b = params["embedding"]
    fea = _prep(elem_fea, elem_weights, W, b)
    nbr_w = [None, None]

    for g in params["graphs"]:
        flat = _flatten_layer_params(g)
        gathered = [_gather_rows(fea, s3, n3) for s3, n3, _ in parts]
        accs = []
        for p, (s3, n3, ef_p) in enumerate(parts):
            self_f, nbr_f = gathered[p]
            res = _edge_compute(self_f, nbr_f, ef_p, nbr_w[p], flat)
            if nbr_w[p] is None:
                w, wm, nbr_w[p] = res
            else:
                w, wm = res
            accs.append(_scatter(wm, w, s3, N))
        (accp0, waccp0), (accp1, waccp1) = accs
        gamma, beta = g["bn"]
        fea = _combine(accp0, accp1, waccp0, waccp1, fea, gamma, beta)

    crys, gates = _crystal(fea, elem_weights,
                           cry_elem_idx.astype(jnp.int32).reshape(N, 1), params)
    return crys, gates
